# Initial kernel scaffold; baseline (speedup 1.0000x reference)
import jax
import jax.numpy as jnp
from jax.experimental import pallas as pl


def kernel(x, edge_index, Wl1, Wr1, b1, Wl2, Wr2, b2, Wl3, Wr3, b3):
    raise NotImplementedError("write your pallas kernel here")


# speedup vs baseline: 2.7546x; 2.7546x over previous
import functools

import jax
import jax.numpy as jnp
from jax import lax
from jax.experimental import pallas as pl
from jax.experimental.pallas import tpu as pltpu
from jax.experimental.pallas import tpu_sc as plsc

N = 10000
E = 160000
D = 256
DH = 128
NACC = 10240
K = 128
NCH = 80
EPAD = 16 * NCH * K
RPT = NACC // 16
L = 16


def _make_agg(compute_deg: bool):
    mesh = plsc.VectorSubcoreMesh(core_axis_name="c", subcore_axis_name="s")
    out_type = [jax.ShapeDtypeStruct((2, NACC, DH), jnp.float32)]
    if compute_deg:
        out_type.append(jax.ShapeDtypeStruct((NACC,), jnp.float32))
    scratch = [
        pltpu.VMEM((NCH, K), jnp.int32),
        pltpu.VMEM((NCH, K), jnp.int32),
        pltpu.VMEM((K, DH), jnp.float32),
        pltpu.VMEM((RPT,), jnp.float32),
        pltpu.VMEM((K,), jnp.float32),
        pltpu.VMEM_SHARED((NACC, DH), jnp.float32),
        pltpu.VMEM_SHARED((NACC,), jnp.float32),
        pltpu.SemaphoreType.DMA,
    ]

    @functools.partial(pl.kernel, mesh=mesh, out_type=out_type,
                       scratch_types=scratch)
    def agg(h2, srcs, dsts, *rest):
        if compute_deg:
            out, deg_out = rest[0], rest[1]
            (src_v, dst_v, rows_v, zero1_v, ones_v,
             acc, dacc, sem) = rest[2:]
        else:
            out = rest[0]
            deg_out = None
            (src_v, dst_v, rows_v, zero1_v, ones_v,
             acc, dacc, sem) = rest[1:]

        c = lax.axis_index("c")
        s = lax.axis_index("s")
        t0 = s * RPT

        def zrow(i, carry):
            for q in range(DH // L):
                rows_v[i, pl.ds(q * L, L)] = jnp.zeros((L,), jnp.float32)
            return carry
        lax.fori_loop(0, K, zrow, 0)

        def z1(i, carry):
            zero1_v[pl.ds(i * L, L)] = jnp.zeros((L,), jnp.float32)
            return carry
        lax.fori_loop(0, RPT // L, z1, 0)

        for q in range(K // L):
            ones_v[pl.ds(q * L, L)] = jnp.full((L,), 1.0, jnp.float32)

        for b in range(RPT // K):
            pltpu.sync_copy(rows_v, acc.at[pl.ds(t0 + b * K, K)])
        pltpu.sync_copy(zero1_v, dacc.at[pl.ds(t0, RPT)])
        plsc.subcore_barrier()

        pltpu.sync_copy(srcs.at[c, pl.ds(s * NCH, NCH)], src_v)
        pltpu.sync_copy(dsts.at[pl.ds(s * NCH, NCH)], dst_v)

        def body(j, carry):
            pltpu.async_copy(h2.at[src_v.at[j]], rows_v, sem).wait()
            pltpu.sync_copy(rows_v, acc.at[dst_v.at[j]], add=True)
            if compute_deg:
                @pl.when(c == 0)
                def _():
                    pltpu.sync_copy(ones_v, dacc.at[dst_v.at[j]], add=True)
            return carry
        lax.fori_loop(0, NCH, body, 0)

        plsc.subcore_barrier()

        pltpu.sync_copy(acc.at[pl.ds(t0, RPT)], out.at[c, pl.ds(t0, RPT)])
        if compute_deg:
            @pl.when(c == 0)
            def _():
                pltpu.sync_copy(dacc.at[pl.ds(t0, RPT)],
                                deg_out.at[pl.ds(t0, RPT)])

    return agg


_agg_deg = _make_agg(True)
_agg = _make_agg(False)


RB = 1024


def _tc_body(s3, h3, deg, Wl, Wr, b2, out, *, split_out):
    sv = s3[...]
    hv = h3[...]
    wl = Wl[...]
    wr = Wr[...]
    aggw = (jnp.dot(sv[0], wl[:DH], preferred_element_type=jnp.float32)
            + jnp.dot(sv[1], wl[DH:], preferred_element_type=jnp.float32))
    scale = 1.0 / jnp.maximum(deg[...], 1.0)
    r = (aggw * scale
         + jnp.dot(hv[0], wr[:DH], preferred_element_type=jnp.float32)
         + jnp.dot(hv[1], wr[DH:], preferred_element_type=jnp.float32)
         + b2[...][0])
    r = jnp.maximum(r, 0.0)
    out[...] = r[None] if split_out else r


def _tc_layer(split_out: bool):
    grid = (NACC // RB, 2)
    in_specs = [
        pl.BlockSpec((2, RB, DH), lambda i, c: (0, i, 0)),
        pl.BlockSpec((2, RB, DH), lambda i, c: (0, i, 0)),
        pl.BlockSpec((RB, 1), lambda i, c: (i, 0)),
        pl.BlockSpec((D, DH), lambda i, c: (0, c)),
        pl.BlockSpec((D, DH), lambda i, c: (0, c)),
        pl.BlockSpec((1, 1, DH), lambda i, c: (c, 0, 0)),
    ]
    if split_out:
        out_shape = jax.ShapeDtypeStruct((2, NACC, DH), jnp.float32)
        out_spec = pl.BlockSpec((1, RB, DH), lambda i, c: (c, i, 0))
    else:
        out_shape = jax.ShapeDtypeStruct((NACC, D), jnp.float32)
        out_spec = pl.BlockSpec((RB, DH), lambda i, c: (i, c))
    body = functools.partial(_tc_body, split_out=split_out)
    return pl.pallas_call(body, grid=grid, in_specs=in_specs,
                          out_specs=out_spec, out_shape=out_shape)


_tc_mid = _tc_layer(True)
_tc_last = _tc_layer(False)


def kernel(x, edge_index, Wl1, Wr1, b1, Wl2, Wr2, b2, Wl3, Wr3, b3):
    src = edge_index[0].astype(jnp.int32)
    dst = edge_index[1].astype(jnp.int32)
    pad = EPAD - E
    srcp = jnp.concatenate([src, jnp.zeros((pad,), jnp.int32)])
    dstp = jnp.concatenate([dst, jnp.full((pad,), N, jnp.int32)])
    srcs = jnp.stack([srcp, srcp + NACC]).reshape(2, 16 * NCH, K)
    dsts = dstp.reshape(16 * NCH, K)

    xp = jnp.pad(x, ((0, NACC - N), (0, 0)))
    h3 = xp.reshape(NACC, 2, DH).transpose(1, 0, 2)

    summed3, deg = _agg_deg(h3.reshape(2 * NACC, DH), srcs, dsts)
    deg2 = deg.reshape(NACC, 1)

    h3 = _tc_mid(summed3, h3, deg2, Wl1, Wr1, b1.reshape(2, 1, DH))
    summed3, = _agg(h3.reshape(2 * NACC, DH), srcs, dsts)
    h3 = _tc_mid(summed3, h3, deg2, Wl2, Wr2, b2.reshape(2, 1, DH))
    summed3, = _agg(h3.reshape(2 * NACC, DH), srcs, dsts)
    out = _tc_last(summed3, h3, deg2, Wl3, Wr3, b3.reshape(2, 1, DH))
    return out[:N]
